# baseline (speedup 1.0000x reference)
import functools
import jax
import jax.numpy as jnp
from jax import lax
from jax.experimental import pallas as pl
from jax.experimental.pallas import tpu as pltpu

N = 10000
T = 8
E = 160000
D_IN = 8
H = 64
NMIX = 7
DI = 256
DS = 10
DCONV = 4
R = 4

BLK = 400
L_SEQ = T + 1


def _silu(v):
    return v * jax.nn.sigmoid(v)


def _layernorm(v, g, b):
    m = jnp.mean(v, axis=-1, keepdims=True)
    c = v - m
    var = jnp.mean(c * c, axis=-1, keepdims=True)
    return c * jax.lax.rsqrt(var + 1e-5) * g + b


def _dense_body(x_ref, agg_ref, Wr_ref, Wn_ref, ball_ref, h0_ref,
                inp_ref, cwT_ref, cb_ref, wdt_ref, dtb_ref, wbc_ref,
                dvec_ref, outp_ref, ln_ref, wm_ref, bm_ref,
                out_ref, mw_ref):
    f32 = jnp.float32
    dot = functools.partial(jnp.dot, preferred_element_type=f32)

    Wr = Wr_ref[...]
    Wn = Wn_ref[...]
    ball = ball_ref[...]
    h0row = h0_ref[...]

    seq1 = [jnp.broadcast_to(h0row, (BLK, H))]
    seq2 = [jnp.broadcast_to(h0row, (BLK, H))]
    for t in range(T):
        xt = x_ref[:, t, :]
        at = agg_ref[t]
        g = dot(xt, Wr) + dot(at, Wn) + ball
        seq1.append(g[:, :H])
        seq2.append(g[:, H:])

    def mamba(seq, m):
        inp = inp_ref[m]
        cwT = cwT_ref[m]
        cb = cb_ref[m]
        wdt = wdt_ref[m]
        dtb = dtb_ref[m]
        wbc = wbc_ref[m]
        dvec = dvec_ref[m]
        outp = outp_ref[m]

        xz = [dot(s, inp) for s in seq]
        xc = [v[:, :DI] for v in xz]
        zg = [v[:, DI:] for v in xz]

        conv = []
        for t in range(L_SEQ):
            acc = cb
            for k in range(DCONV):
                tt = t - (DCONV - 1) + k
                if tt >= 0:
                    acc = acc + xc[tt] * cwT[k:k + 1, :]
            conv.append(_silu(acc))

        h = [jnp.zeros((BLK, DI), f32) for _ in range(DS)]
        outseq = []
        for t in range(L_SEQ):
            xct = conv[t]
            dt = jax.nn.softplus(dot(xct, wdt) + dtb)
            bc = dot(xct, wbc)
            Eb = jnp.exp(-dt)
            u = dt * xct
            dAc = Eb
            yt = jnp.zeros((BLK, DI), f32)
            for s in range(DS):
                hs = dAc * h[s] + u * bc[:, s:s + 1]
                h[s] = hs
                yt = yt + hs * bc[:, DS + s:DS + s + 1]
                if s < DS - 1:
                    dAc = dAc * Eb
            y = (yt + xct * dvec) * _silu(zg[t])
            outseq.append(dot(y, outp))
        return outseq

    ln3g = ln_ref[0:1, :]
    ln3b = ln_ref[1:2, :]
    ln4g = ln_ref[2:3, :]
    ln4b = ln_ref[3:4, :]

    m3out = mamba(seq1, 0)
    a_seq = [jnp.tanh(_layernorm(v, ln3g, ln3b)) for v in m3out]
    seqb = [seq1[T - t] * seq2[t] for t in range(L_SEQ)]
    m4out = mamba(seqb, 1)
    b_seq = [jnp.tanh(_layernorm(v, ln4g, ln4b)) for v in m4out]
    m5out = mamba([a_seq[t] + b_seq[t] for t in range(L_SEQ)], 2)

    for t in range(L_SEQ):
        out_ref[:, t, :] = m5out[t]
    mw_ref[...] = dot(jnp.tanh(m5out[T]), wm_ref[...]) + bm_ref[...]


def _dense_call(x, agg, dense_params):
    (Wr, Wn, ball, h0, inp_all, cwT_all, cb_all, wdt_all, dtb_all,
     wbc_all, dvec_all, outp_all, ln_all, wm, bm) = dense_params
    nblocks = N // BLK
    rep = lambda *shape: pl.BlockSpec(shape, lambda i: (0,) * len(shape))
    out_shapes = (
        jax.ShapeDtypeStruct((N, L_SEQ, H), jnp.float32),
        jax.ShapeDtypeStruct((N, NMIX), jnp.float32),
    )
    return pl.pallas_call(
        _dense_body,
        grid=(nblocks,),
        in_specs=[
            pl.BlockSpec((BLK, T, D_IN), lambda i: (i, 0, 0)),
            pl.BlockSpec((T, BLK, D_IN), lambda i: (0, i, 0)),
            rep(D_IN, 2 * H), rep(D_IN, 2 * H), rep(1, 2 * H), rep(1, H),
            rep(3, H, 2 * DI), rep(3, DCONV, DI), rep(3, 1, DI),
            rep(3, DI, DI), rep(3, 1, DI), rep(3, DI, 2 * DS),
            rep(3, 1, DI), rep(3, DI, H), rep(4, H), rep(H, NMIX),
            rep(1, NMIX),
        ],
        out_specs=[
            pl.BlockSpec((BLK, L_SEQ, H), lambda i: (i, 0, 0)),
            pl.BlockSpec((BLK, NMIX), lambda i: (i, 0)),
        ],
        out_shape=out_shapes,
    )(x, agg, Wr, Wn, ball, h0, inp_all, cwT_all, cb_all, wdt_all,
      dtb_all, wbc_all, dvec_all, outp_all, ln_all, wm, bm)


def _prep_dense_params(params):
    f32 = jnp.float32
    Wr = params['Wr'].astype(f32)
    Wn = params['Wn'].astype(f32)
    ball = (params['bg'] + params['bias']).reshape(1, 2 * H)
    h0 = params['h0'].reshape(1, H)
    ms = [params['m3'], params['m4'], params['m5']]
    inp_all = jnp.stack([p['in_proj'] for p in ms])
    cwT_all = jnp.stack([p['conv_w'].T for p in ms])
    cb_all = jnp.stack([p['conv_b'].reshape(1, DI) for p in ms])
    wdt_all = jnp.stack([p['x_proj'][:, :R] @ p['dt_w'] for p in ms])
    dtb_all = jnp.stack([p['dt_b'].reshape(1, DI) for p in ms])
    wbc_all = jnp.stack([p['x_proj'][:, R:] for p in ms])
    dvec_all = jnp.stack([p['D'].reshape(1, DI) for p in ms])
    outp_all = jnp.stack([p['out_proj'] for p in ms])
    ln_all = jnp.stack([params['ln3_g'], params['ln3_b'],
                        params['ln4_g'], params['ln4_b']])
    wm = params['Wm']
    bm = params['bm'].reshape(1, NMIX)
    return (Wr, Wn, ball, h0, inp_all, cwT_all, cb_all, wdt_all,
            dtb_all, wbc_all, dvec_all, outp_all, ln_all, wm, bm)


def _edge_agg(x, edge_index, edge_features):
    src = edge_index[:, 0, :]
    dst = edge_index[:, 1, :]
    ew = edge_features[:, :, 0]
    xT = jnp.transpose(x, (1, 0, 2))

    def per_t(xt, s, d, w):
        msg = xt[s] * w[:, None]
        return jax.ops.segment_sum(msg, d, num_segments=N)

    return jax.vmap(per_t)(xT, src, dst, ew)


@jax.jit
def kernel(x, edge_index, edge_features, params):
    agg = _edge_agg(x, edge_index, edge_features)
    dense_params = _prep_dense_params(params)
    return _dense_call(x, agg, dense_params)


# speedup vs baseline: 3.4598x; 3.4598x over previous
import functools
import jax
import jax.numpy as jnp
from jax import lax
from jax.experimental import pallas as pl
from jax.experimental.pallas import tpu as pltpu

N = 10000
T = 8
E = 160000
D_IN = 8
H = 64
NMIX = 7
DI = 256
DS = 10
DCONV = 4
R = 4

BLK = 512
N_PAD = 10240
L_SEQ = T + 1


def _silu(v):
    return v * jax.nn.sigmoid(v)


def _layernorm_t(v, g, b):
    m = jnp.mean(v, axis=0, keepdims=True)
    c = v - m
    var = jnp.mean(c * c, axis=0, keepdims=True)
    return c * jax.lax.rsqrt(var + 1e-5) * g + b


def _dense_body(x_ref, agg_ref, WrT_ref, WnT_ref, ball_ref, h0_ref,
                inpT_ref, cw_ref, cb_ref, wdtT_ref, dtb_ref, wbcT_ref,
                dvec_ref, outpT_ref, ln_ref, wmT_ref, bm_ref,
                out_ref, mw_ref):
    f32 = jnp.float32
    dot = functools.partial(jnp.dot, preferred_element_type=f32)
    bcast = lambda col: jnp.broadcast_to(col, (col.shape[0], BLK))

    WrT = WrT_ref[...]
    WnT = WnT_ref[...]
    ball = bcast(ball_ref[...])
    h0b = bcast(h0_ref[...])

    seq1 = [h0b]
    seq2 = [h0b]
    for t in range(T):
        xt = x_ref[t]
        at = agg_ref[t]
        g = dot(WrT, xt) + dot(WnT, at) + ball
        seq1.append(g[:H, :])
        seq2.append(g[H:, :])

    def mamba(seq, m):
        inpT = inpT_ref[m]
        cwb = [bcast(cw_ref[m, :, k:k + 1]) for k in range(DCONV)]
        cbb = bcast(cb_ref[m])
        wdtT = wdtT_ref[m]
        dtbb = bcast(dtb_ref[m])
        wbcT = wbcT_ref[m]
        dvb = bcast(dvec_ref[m])
        outpT = outpT_ref[m]

        xz = [dot(inpT, s) for s in seq]
        xc = [v[:DI, :] for v in xz]
        zg = [v[DI:, :] for v in xz]

        conv = []
        for t in range(L_SEQ):
            acc = cbb
            for k in range(DCONV):
                tt = t - (DCONV - 1) + k
                if tt >= 0:
                    acc = acc + xc[tt] * cwb[k]
            conv.append(_silu(acc))

        h = [jnp.zeros((DI, BLK), f32) for _ in range(DS)]
        outseq = []
        for t in range(L_SEQ):
            xct = conv[t]
            dt = jax.nn.softplus(dot(wdtT, xct) + dtbb)
            bcT = dot(wbcT, xct)
            Eb = jnp.exp(-dt)
            u = dt * xct
            dAc = Eb
            yt = jnp.zeros((DI, BLK), f32)
            for s in range(DS):
                hs = dAc * h[s] + u * bcT[s:s + 1, :]
                h[s] = hs
                yt = yt + hs * bcT[DS + s:DS + s + 1, :]
                if s < DS - 1:
                    dAc = dAc * Eb
            y = (yt + xct * dvb) * _silu(zg[t])
            outseq.append(dot(outpT, y))
        return outseq

    ln3g = bcast(ln_ref[0:1, :].reshape(H, 1))
    ln3b = bcast(ln_ref[1:2, :].reshape(H, 1))
    ln4g = bcast(ln_ref[2:3, :].reshape(H, 1))
    ln4b = bcast(ln_ref[3:4, :].reshape(H, 1))

    m3out = mamba(seq1, 0)
    a_seq = [jnp.tanh(_layernorm_t(v, ln3g, ln3b)) for v in m3out]
    seqb = [seq1[T - t] * seq2[t] for t in range(L_SEQ)]
    m4out = mamba(seqb, 1)
    b_seq = [jnp.tanh(_layernorm_t(v, ln4g, ln4b)) for v in m4out]
    m5out = mamba([a_seq[t] + b_seq[t] for t in range(L_SEQ)], 2)

    for t in range(L_SEQ):
        out_ref[t] = m5out[t]
    mw_ref[...] = dot(wmT_ref[...], jnp.tanh(m5out[T])) + bcast(bm_ref[...])


def _dense_call(xT, aggT, dense_params):
    (WrT, WnT, ball, h0, inpT_all, cw_all, cb_all, wdtT_all, dtb_all,
     wbcT_all, dvec_all, outpT_all, ln_all, wmT, bm) = dense_params
    nblocks = N_PAD // BLK
    rep = lambda *shape: pl.BlockSpec(shape, lambda i: (0,) * len(shape))
    out_shapes = (
        jax.ShapeDtypeStruct((L_SEQ, H, N_PAD), jnp.float32),
        jax.ShapeDtypeStruct((NMIX, N_PAD), jnp.float32),
    )
    return pl.pallas_call(
        _dense_body,
        grid=(nblocks,),
        in_specs=[
            pl.BlockSpec((T, D_IN, BLK), lambda i: (0, 0, i)),
            pl.BlockSpec((T, D_IN, BLK), lambda i: (0, 0, i)),
            rep(2 * H, D_IN), rep(2 * H, D_IN), rep(2 * H, 1), rep(H, 1),
            rep(3, 2 * DI, H), rep(3, DI, DCONV), rep(3, DI, 1),
            rep(3, DI, DI), rep(3, DI, 1), rep(3, 2 * DS, DI),
            rep(3, DI, 1), rep(3, H, DI), rep(4, H), rep(NMIX, H),
            rep(NMIX, 1),
        ],
        out_specs=[
            pl.BlockSpec((L_SEQ, H, BLK), lambda i: (0, 0, i)),
            pl.BlockSpec((NMIX, BLK), lambda i: (0, i)),
        ],
        out_shape=out_shapes,
    )(xT, aggT, WrT, WnT, ball, h0, inpT_all, cw_all, cb_all, wdtT_all,
      dtb_all, wbcT_all, dvec_all, outpT_all, ln_all, wmT, bm)


def _prep_dense_params(params):
    f32 = jnp.float32
    WrT = params['Wr'].T.astype(f32)
    WnT = params['Wn'].T.astype(f32)
    ball = (params['bg'] + params['bias']).reshape(2 * H, 1)
    h0 = params['h0'].reshape(H, 1)
    ms = [params['m3'], params['m4'], params['m5']]
    inpT_all = jnp.stack([p['in_proj'].T for p in ms])
    cw_all = jnp.stack([p['conv_w'] for p in ms])
    cb_all = jnp.stack([p['conv_b'].reshape(DI, 1) for p in ms])
    wdtT_all = jnp.stack([(p['x_proj'][:, :R] @ p['dt_w']).T for p in ms])
    dtb_all = jnp.stack([p['dt_b'].reshape(DI, 1) for p in ms])
    wbcT_all = jnp.stack([p['x_proj'][:, R:].T for p in ms])
    dvec_all = jnp.stack([p['D'].reshape(DI, 1) for p in ms])
    outpT_all = jnp.stack([p['out_proj'].T for p in ms])
    ln_all = jnp.stack([params['ln3_g'], params['ln3_b'],
                        params['ln4_g'], params['ln4_b']])
    wmT = params['Wm'].T
    bm = params['bm'].reshape(NMIX, 1)
    return (WrT, WnT, ball, h0, inpT_all, cw_all, cb_all, wdtT_all,
            dtb_all, wbcT_all, dvec_all, outpT_all, ln_all, wmT, bm)


def _edge_agg(x, edge_index, edge_features):
    xT = jnp.transpose(x, (1, 0, 2))
    aggs = []
    for t in range(T):
        msg = xT[t][edge_index[t, 0]] * edge_features[t]
        aggs.append(jax.ops.segment_sum(msg, edge_index[t, 1],
                                        num_segments=N))
    return jnp.stack(aggs)


@jax.jit
def kernel(x, edge_index, edge_features, params):
    agg = _edge_agg(x, edge_index, edge_features)
    dense_params = _prep_dense_params(params)
    pad = [(0, 0), (0, 0), (0, N_PAD - N)]
    xT = jnp.pad(jnp.transpose(x, (1, 2, 0)), pad)
    aggT = jnp.pad(jnp.transpose(agg, (0, 2, 1)), pad)
    outT, mwT = _dense_call(xT, aggT, dense_params)
    out = jnp.transpose(outT, (2, 0, 1))[:N]
    mw = mwT.T[:N]
    return (out, mw)


# speedup vs baseline: 16.4725x; 4.7611x over previous
import functools
import jax
import jax.numpy as jnp
from jax import lax
from jax.experimental import pallas as pl
from jax.experimental.pallas import tpu as pltpu
from jax.experimental.pallas import tpu_sc as plsc

N = 10000
T = 8
E = 160000
D_IN = 8
H = 64
NMIX = 7
DI = 256
DS = 10
DCONV = 4
R = 4

BLK = 512
N_PAD = 10240
L_SEQ = T + 1


def _silu(v):
    return v * jax.nn.sigmoid(v)


def _layernorm_t(v, g, b):
    m = jnp.mean(v, axis=0, keepdims=True)
    c = v - m
    var = jnp.mean(c * c, axis=0, keepdims=True)
    return c * jax.lax.rsqrt(var + 1e-5) * g + b


def _dense_body(x_ref, agg_ref, WrT_ref, WnT_ref, ball_ref, h0_ref,
                inpT_ref, cw_ref, cb_ref, wdtT_ref, dtb_ref, wbcT_ref,
                dvec_ref, outpT_ref, ln_ref, wmT_ref, bm_ref,
                out_ref, mw_ref):
    f32 = jnp.float32
    dot = functools.partial(jnp.dot, preferred_element_type=f32)
    bcast = lambda col: jnp.broadcast_to(col, (col.shape[0], BLK))

    WrT = WrT_ref[...]
    WnT = WnT_ref[...]
    ball = bcast(ball_ref[...])
    h0b = bcast(h0_ref[...])

    seq1 = [h0b]
    seq2 = [h0b]
    for t in range(T):
        xt = x_ref[t]
        at = agg_ref[0, t] + agg_ref[1, t]
        g = dot(WrT, xt) + dot(WnT, at) + ball
        seq1.append(g[:H, :])
        seq2.append(g[H:, :])

    def mamba(seq, m):
        inpT = inpT_ref[m]
        cwb = [bcast(cw_ref[m, :, k:k + 1]) for k in range(DCONV)]
        cbb = bcast(cb_ref[m])
        wdtT = wdtT_ref[m]
        dtbb = bcast(dtb_ref[m])
        wbcT = wbcT_ref[m]
        dvb = bcast(dvec_ref[m])
        outpT = outpT_ref[m]

        xz = [dot(inpT, s) for s in seq]
        xc = [v[:DI, :] for v in xz]
        zg = [v[DI:, :] for v in xz]

        conv = []
        for t in range(L_SEQ):
            acc = cbb
            for k in range(DCONV):
                tt = t - (DCONV - 1) + k
                if tt >= 0:
                    acc = acc + xc[tt] * cwb[k]
            conv.append(_silu(acc))

        h = [jnp.zeros((DI, BLK), f32) for _ in range(DS)]
        outseq = []
        for t in range(L_SEQ):
            xct = conv[t]
            dt = jax.nn.softplus(dot(wdtT, xct) + dtbb)
            bcT = dot(wbcT, xct)
            Eb = jnp.exp(-dt)
            u = dt * xct
            dAc = Eb
            yt = jnp.zeros((DI, BLK), f32)
            for s in range(DS):
                hs = dAc * h[s] + u * bcT[s:s + 1, :]
                h[s] = hs
                yt = yt + hs * bcT[DS + s:DS + s + 1, :]
                if s < DS - 1:
                    dAc = dAc * Eb
            y = (yt + xct * dvb) * _silu(zg[t])
            outseq.append(dot(outpT, y))
        return outseq

    ln3g = bcast(ln_ref[0:1, :].reshape(H, 1))
    ln3b = bcast(ln_ref[1:2, :].reshape(H, 1))
    ln4g = bcast(ln_ref[2:3, :].reshape(H, 1))
    ln4b = bcast(ln_ref[3:4, :].reshape(H, 1))

    m3out = mamba(seq1, 0)
    a_seq = [jnp.tanh(_layernorm_t(v, ln3g, ln3b)) for v in m3out]
    seqb = [seq1[T - t] * seq2[t] for t in range(L_SEQ)]
    m4out = mamba(seqb, 1)
    b_seq = [jnp.tanh(_layernorm_t(v, ln4g, ln4b)) for v in m4out]
    m5out = mamba([a_seq[t] + b_seq[t] for t in range(L_SEQ)], 2)

    for t in range(L_SEQ):
        out_ref[t] = m5out[t]
    mw_ref[...] = dot(wmT_ref[...], jnp.tanh(m5out[T])) + bcast(bm_ref[...])


def _dense_call(xT, aggT, dense_params):
    (WrT, WnT, ball, h0, inpT_all, cw_all, cb_all, wdtT_all, dtb_all,
     wbcT_all, dvec_all, outpT_all, ln_all, wmT, bm) = dense_params
    nblocks = N_PAD // BLK
    rep = lambda *shape: pl.BlockSpec(shape, lambda i: (0,) * len(shape))
    out_shapes = (
        jax.ShapeDtypeStruct((L_SEQ, H, N_PAD), jnp.float32),
        jax.ShapeDtypeStruct((NMIX, N_PAD), jnp.float32),
    )
    return pl.pallas_call(
        _dense_body,
        grid=(nblocks,),
        in_specs=[
            pl.BlockSpec((T, D_IN, BLK), lambda i: (0, 0, i)),
            pl.BlockSpec((2, T, D_IN, BLK), lambda i: (0, 0, 0, i)),
            rep(2 * H, D_IN), rep(2 * H, D_IN), rep(2 * H, 1), rep(H, 1),
            rep(3, 2 * DI, H), rep(3, DI, DCONV), rep(3, DI, 1),
            rep(3, DI, DI), rep(3, DI, 1), rep(3, 2 * DS, DI),
            rep(3, DI, 1), rep(3, H, DI), rep(4, H), rep(NMIX, H),
            rep(NMIX, 1),
        ],
        out_specs=[
            pl.BlockSpec((L_SEQ, H, BLK), lambda i: (0, 0, i)),
            pl.BlockSpec((NMIX, BLK), lambda i: (0, i)),
        ],
        out_shape=out_shapes,
    )(xT, aggT, WrT, WnT, ball, h0, inpT_all, cw_all, cb_all, wdtT_all,
      dtb_all, wbcT_all, dvec_all, outpT_all, ln_all, wmT, bm)


def _prep_dense_params(params):
    f32 = jnp.float32
    WrT = params['Wr'].T.astype(f32)
    WnT = params['Wn'].T.astype(f32)
    ball = (params['bg'] + params['bias']).reshape(2 * H, 1)
    h0 = params['h0'].reshape(H, 1)
    ms = [params['m3'], params['m4'], params['m5']]
    inpT_all = jnp.stack([p['in_proj'].T for p in ms])
    cw_all = jnp.stack([p['conv_w'] for p in ms])
    cb_all = jnp.stack([p['conv_b'].reshape(DI, 1) for p in ms])
    wdtT_all = jnp.stack([(p['x_proj'][:, :R] @ p['dt_w']).T for p in ms])
    dtb_all = jnp.stack([p['dt_b'].reshape(DI, 1) for p in ms])
    wbcT_all = jnp.stack([p['x_proj'][:, R:].T for p in ms])
    dvec_all = jnp.stack([p['D'].reshape(DI, 1) for p in ms])
    outpT_all = jnp.stack([p['out_proj'].T for p in ms])
    ln_all = jnp.stack([params['ln3_g'], params['ln3_b'],
                        params['ln4_g'], params['ln4_b']])
    wmT = params['Wm'].T
    bm = params['bm'].reshape(NMIX, 1)
    return (WrT, WnT, ball, h0, inpT_all, cw_all, cb_all, wdtT_all,
            dtb_all, wbcT_all, dvec_all, outpT_all, ln_all, wmT, bm)


_NC = 2
_NS = 16
_EPW = 5120
E_PAD = _NC * _NS * _EPW
_SC_C = 1024
_SC_KD = _SC_C // 128
_NCHUNK = _EPW // _SC_C
_NROW = N_PAD // _NS


def _edge_body(xT_hbm, src_hbm, dst_hbm, ew_hbm, zrow_hbm, out_hbm,
               a0, a1, a2, a3, a4, a5, a6, a7, x_sh,
               src_v, dst_v, ew_v, rows_v, gsem, ssem):
    aggs = [a0, a1, a2, a3, a4, a5, a6, a7]
    cid = lax.axis_index("c")
    sid = lax.axis_index("s")
    wid = cid * _NS + sid

    rslab = pl.ds(sid * _NROW, _NROW)
    for t in range(T):
        pltpu.sync_copy(zrow_hbm, aggs[t].at[rslab])

    lane = lax.broadcasted_iota(jnp.int32, (16,), 0)
    half = lane // 8
    colv = lane % 8

    for t in range(T):
        aggt = aggs[t]
        pltpu.sync_copy(xT_hbm.at[t, rslab], x_sh.at[rslab])
        plsc.subcore_barrier()
        for ci in range(_NCHUNK):
            roff = wid * (_EPW // 128) + ci * _SC_KD
            pltpu.sync_copy(src_hbm.at[t, pl.ds(roff, _SC_KD)], src_v)
            pltpu.sync_copy(ew_hbm.at[t, pl.ds(roff, _SC_KD)], ew_v)
            pltpu.sync_copy(dst_hbm.at[t, pl.ds(roff, _SC_KD)], dst_v)
            gds = [
                pltpu.async_copy(x_sh.at[src_v.at[j]],
                                 rows_v.at[pl.ds(j * 128, 128)], gsem)
                for j in range(_SC_KD)
            ]
            for d in gds:
                d.wait()

            @plsc.parallel_loop(0, _SC_C // 2, 1, unroll=8)
            def _mul(i):
                rew = jnp.broadcast_to(i // 64, (16,))
                cew = (i % 64) * 2 + half
                ridx = 2 * i + half
                ew16 = plsc.load_gather(ew_v, [rew, cew])
                v16 = plsc.load_gather(rows_v, [ridx, colv])
                plsc.store_scatter(rows_v, [ridx, colv], v16 * ew16)

            sds = [
                pltpu.async_copy(rows_v.at[pl.ds(j * 128, 128)],
                                 aggt.at[dst_v.at[j]], ssem, add=True)
                for j in range(_SC_KD)
            ]
            for d in sds:
                d.wait()
        plsc.subcore_barrier()

    for t in range(T):
        pltpu.sync_copy(aggs[t].at[rslab], out_hbm.at[cid, t, rslab])


def _edge_call(xP, srcp, dstp, ewp, zrow):
    mesh = plsc.VectorSubcoreMesh(core_axis_name="c", subcore_axis_name="s")
    f = pl.kernel(
        _edge_body,
        out_type=jax.ShapeDtypeStruct((_NC, T, N_PAD, D_IN), jnp.float32),
        mesh=mesh,
        compiler_params=pltpu.CompilerParams(use_tc_tiling_on_sc=False, needs_layout_passes=False),
        scratch_types=(
            [pltpu.VMEM_SHARED((N_PAD, D_IN), jnp.float32)
             for _ in range(T + 1)]
            + [pltpu.VMEM((_SC_KD, 128), jnp.int32),
               pltpu.VMEM((_SC_KD, 128), jnp.int32),
               pltpu.VMEM((_SC_KD, 128), jnp.float32),
               pltpu.VMEM((_SC_C, D_IN), jnp.float32),
               pltpu.SemaphoreType.DMA,
               pltpu.SemaphoreType.DMA]
        ),
    )
    return f(xP, srcp, dstp, ewp, zrow)


def _edge_agg(x, edge_index, edge_features):
    i32 = jnp.int32
    src = edge_index[:, 0, :].astype(i32)
    dst = edge_index[:, 1, :].astype(i32)
    ew = edge_features[:, :, 0]
    padE = [(0, 0), (0, E_PAD - E)]
    sh3 = (T, E_PAD // 128, 128)
    srcp = jnp.pad(src, padE).reshape(sh3)
    dstp = jnp.pad(dst, padE).reshape(sh3)
    ewp = jnp.pad(ew, padE).reshape(sh3)
    xP = jnp.pad(jnp.transpose(x, (1, 0, 2)),
                 [(0, 0), (0, N_PAD - N), (0, 0)])
    zrow = jnp.zeros((_NROW, D_IN), jnp.float32)
    return _edge_call(xP, srcp, dstp, ewp, zrow)


@jax.jit
def kernel(x, edge_index, edge_features, params):
    agg = _edge_agg(x, edge_index, edge_features)
    dense_params = _prep_dense_params(params)
    pad = [(0, 0), (0, 0), (0, N_PAD - N)]
    xT = jnp.pad(jnp.transpose(x, (1, 2, 0)), pad)
    aggT = jnp.transpose(agg, (0, 1, 3, 2))
    outT, mwT = _dense_call(xT, aggT, dense_params)
    out = jnp.transpose(outT, (2, 0, 1))[:N]
    mw = mwT.T[:N]
    return (out, mw)


# speedup vs baseline: 17.6206x; 1.0697x over previous
import functools
import jax
import jax.numpy as jnp
from jax import lax
from jax.experimental import pallas as pl
from jax.experimental.pallas import tpu as pltpu
from jax.experimental.pallas import tpu_sc as plsc

N = 10000
T = 8
E = 160000
D_IN = 8
H = 64
NMIX = 7
DI = 256
DS = 10
DCONV = 4
R = 4

BLK = 1024
N_PAD = 10240
L_SEQ = T + 1


def _silu(v):
    return v * jax.nn.sigmoid(v)


def _layernorm_t(v, g, b):
    m = jnp.mean(v, axis=0, keepdims=True)
    c = v - m
    var = jnp.mean(c * c, axis=0, keepdims=True)
    return c * jax.lax.rsqrt(var + 1e-5) * g + b


def _dense_body(x_ref, agg_ref, WrT_ref, WnT_ref, ball_ref, h0_ref,
                inpT_ref, cw_ref, cb_ref, wdtT_ref, dtb_ref, wbcT_ref,
                dvec_ref, outpT_ref, ln_ref, wmT_ref, bm_ref,
                out_ref, mw_ref):
    f32 = jnp.float32
    dot = functools.partial(jnp.dot, preferred_element_type=f32)
    bcast = lambda col: jnp.broadcast_to(col, (col.shape[0], BLK))

    WrT = WrT_ref[...]
    WnT = WnT_ref[...]
    ball = bcast(ball_ref[...])
    h0b = bcast(h0_ref[...])

    seq1 = [h0b]
    seq2 = [h0b]
    for t in range(T):
        xt = x_ref[t]
        at = agg_ref[0, t] + agg_ref[1, t]
        g = dot(WrT, xt) + dot(WnT, at) + ball
        seq1.append(g[:H, :])
        seq2.append(g[H:, :])

    def mamba(seq, m):
        inpT = inpT_ref[m]
        cwb = [bcast(cw_ref[m, :, k:k + 1]) for k in range(DCONV)]
        cbb = bcast(cb_ref[m])
        wdtT = wdtT_ref[m]
        dtbb = bcast(dtb_ref[m])
        wbcT = wbcT_ref[m]
        dvb = bcast(dvec_ref[m])
        outpT = outpT_ref[m]

        xz = [dot(inpT, s) for s in seq]
        xc = [v[:DI, :] for v in xz]
        zg = [v[DI:, :] for v in xz]

        conv = []
        for t in range(L_SEQ):
            acc = cbb
            for k in range(DCONV):
                tt = t - (DCONV - 1) + k
                if tt >= 0:
                    acc = acc + xc[tt] * cwb[k]
            conv.append(_silu(acc))

        h = [jnp.zeros((DI, BLK), f32) for _ in range(DS)]
        outseq = []
        for t in range(L_SEQ):
            xct = conv[t]
            dt = jax.nn.softplus(dot(wdtT, xct) + dtbb)
            bcT = dot(wbcT, xct)
            Eb = jnp.exp(-dt)
            u = dt * xct
            dAc = Eb
            yt = jnp.zeros((DI, BLK), f32)
            for s in range(DS):
                hs = dAc * h[s] + u * bcT[s:s + 1, :]
                h[s] = hs
                yt = yt + hs * bcT[DS + s:DS + s + 1, :]
                if s < DS - 1:
                    dAc = dAc * Eb
            y = (yt + xct * dvb) * _silu(zg[t])
            outseq.append(dot(outpT, y))
        return outseq

    ln3g = bcast(ln_ref[0:1, :].reshape(H, 1))
    ln3b = bcast(ln_ref[1:2, :].reshape(H, 1))
    ln4g = bcast(ln_ref[2:3, :].reshape(H, 1))
    ln4b = bcast(ln_ref[3:4, :].reshape(H, 1))

    m3out = mamba(seq1, 0)
    a_seq = [jnp.tanh(_layernorm_t(v, ln3g, ln3b)) for v in m3out]
    seqb = [seq1[T - t] * seq2[t] for t in range(L_SEQ)]
    m4out = mamba(seqb, 1)
    b_seq = [jnp.tanh(_layernorm_t(v, ln4g, ln4b)) for v in m4out]
    m5out = mamba([a_seq[t] + b_seq[t] for t in range(L_SEQ)], 2)

    for t in range(L_SEQ):
        out_ref[t] = m5out[t]
    mw_ref[...] = dot(wmT_ref[...], jnp.tanh(m5out[T])) + bcast(bm_ref[...])


def _dense_call(xT, aggT, dense_params):
    (WrT, WnT, ball, h0, inpT_all, cw_all, cb_all, wdtT_all, dtb_all,
     wbcT_all, dvec_all, outpT_all, ln_all, wmT, bm) = dense_params
    nblocks = N_PAD // BLK
    rep = lambda *shape: pl.BlockSpec(shape, lambda i: (0,) * len(shape))
    out_shapes = (
        jax.ShapeDtypeStruct((L_SEQ, H, N_PAD), jnp.float32),
        jax.ShapeDtypeStruct((NMIX, N_PAD), jnp.float32),
    )
    return pl.pallas_call(
        _dense_body,
        grid=(nblocks,),
        in_specs=[
            pl.BlockSpec((T, D_IN, BLK), lambda i: (0, 0, i)),
            pl.BlockSpec((2, T, D_IN, BLK), lambda i: (0, 0, 0, i)),
            rep(2 * H, D_IN), rep(2 * H, D_IN), rep(2 * H, 1), rep(H, 1),
            rep(3, 2 * DI, H), rep(3, DI, DCONV), rep(3, DI, 1),
            rep(3, DI, DI), rep(3, DI, 1), rep(3, 2 * DS, DI),
            rep(3, DI, 1), rep(3, H, DI), rep(4, H), rep(NMIX, H),
            rep(NMIX, 1),
        ],
        out_specs=[
            pl.BlockSpec((L_SEQ, H, BLK), lambda i: (0, 0, i)),
            pl.BlockSpec((NMIX, BLK), lambda i: (0, i)),
        ],
        out_shape=out_shapes,
    )(xT, aggT, WrT, WnT, ball, h0, inpT_all, cw_all, cb_all, wdtT_all,
      dtb_all, wbcT_all, dvec_all, outpT_all, ln_all, wmT, bm)


def _prep_dense_params(params):
    f32 = jnp.float32
    WrT = params['Wr'].T.astype(f32)
    WnT = params['Wn'].T.astype(f32)
    ball = (params['bg'] + params['bias']).reshape(2 * H, 1)
    h0 = params['h0'].reshape(H, 1)
    ms = [params['m3'], params['m4'], params['m5']]
    inpT_all = jnp.stack([p['in_proj'].T for p in ms])
    cw_all = jnp.stack([p['conv_w'] for p in ms])
    cb_all = jnp.stack([p['conv_b'].reshape(DI, 1) for p in ms])
    wdtT_all = jnp.stack([(p['x_proj'][:, :R] @ p['dt_w']).T for p in ms])
    dtb_all = jnp.stack([p['dt_b'].reshape(DI, 1) for p in ms])
    wbcT_all = jnp.stack([p['x_proj'][:, R:].T for p in ms])
    dvec_all = jnp.stack([p['D'].reshape(DI, 1) for p in ms])
    outpT_all = jnp.stack([p['out_proj'].T for p in ms])
    ln_all = jnp.stack([params['ln3_g'], params['ln3_b'],
                        params['ln4_g'], params['ln4_b']])
    wmT = params['Wm'].T
    bm = params['bm'].reshape(NMIX, 1)
    return (WrT, WnT, ball, h0, inpT_all, cw_all, cb_all, wdtT_all,
            dtb_all, wbcT_all, dvec_all, outpT_all, ln_all, wmT, bm)


_NC = 2
_NS = 16
_EPW = 5120
E_PAD = _NC * _NS * _EPW
_SC_C = 1024
_SC_KD = _SC_C // 128
_NCHUNK = _EPW // _SC_C
_NROW = N_PAD // _NS


def _edge_body(xT_hbm, src_hbm, dst_hbm, ew_hbm, zrow_hbm, out_hbm,
               a0, a1, a2, a3, a4, a5, a6, a7, x_sh,
               src_v, dst_v, ew_v, rows_v, gsem, ssem):
    aggs = [a0, a1, a2, a3, a4, a5, a6, a7]
    cid = lax.axis_index("c")
    sid = lax.axis_index("s")
    wid = cid * _NS + sid

    rslab = pl.ds(sid * _NROW, _NROW)
    for t in range(T):
        pltpu.sync_copy(zrow_hbm, aggs[t].at[rslab])

    lane = lax.broadcasted_iota(jnp.int32, (16,), 0)
    half = lane // 8
    colv = lane % 8

    for t in range(T):
        aggt = aggs[t]
        pltpu.sync_copy(xT_hbm.at[t, rslab], x_sh.at[rslab])
        plsc.subcore_barrier()
        for ci in range(_NCHUNK):
            roff = wid * (_EPW // 128) + ci * _SC_KD
            pltpu.sync_copy(src_hbm.at[t, pl.ds(roff, _SC_KD)], src_v)
            pltpu.sync_copy(ew_hbm.at[t, pl.ds(roff, _SC_KD)], ew_v)
            pltpu.sync_copy(dst_hbm.at[t, pl.ds(roff, _SC_KD)], dst_v)
            gds = [
                pltpu.async_copy(x_sh.at[src_v.at[j]],
                                 rows_v.at[pl.ds(j * 128, 128)], gsem)
                for j in range(_SC_KD)
            ]
            for d in gds:
                d.wait()

            @plsc.parallel_loop(0, _SC_C // 2, 1, unroll=8)
            def _mul(i):
                rew = jnp.broadcast_to(i // 64, (16,))
                cew = (i % 64) * 2 + half
                ridx = 2 * i + half
                ew16 = plsc.load_gather(ew_v, [rew, cew])
                v16 = plsc.load_gather(rows_v, [ridx, colv])
                plsc.store_scatter(rows_v, [ridx, colv], v16 * ew16)

            sds = [
                pltpu.async_copy(rows_v.at[pl.ds(j * 128, 128)],
                                 aggt.at[dst_v.at[j]], ssem, add=True)
                for j in range(_SC_KD)
            ]
            for d in sds:
                d.wait()
        plsc.subcore_barrier()

    for t in range(T):
        pltpu.sync_copy(aggs[t].at[rslab], out_hbm.at[cid, t, rslab])


def _edge_call(xP, srcp, dstp, ewp, zrow):
    mesh = plsc.VectorSubcoreMesh(core_axis_name="c", subcore_axis_name="s")
    f = pl.kernel(
        _edge_body,
        out_type=jax.ShapeDtypeStruct((_NC, T, N_PAD, D_IN), jnp.float32),
        mesh=mesh,
        compiler_params=pltpu.CompilerParams(use_tc_tiling_on_sc=False, needs_layout_passes=False),
        scratch_types=(
            [pltpu.VMEM_SHARED((N_PAD, D_IN), jnp.float32)
             for _ in range(T + 1)]
            + [pltpu.VMEM((_SC_KD, 128), jnp.int32),
               pltpu.VMEM((_SC_KD, 128), jnp.int32),
               pltpu.VMEM((_SC_KD, 128), jnp.float32),
               pltpu.VMEM((_SC_C, D_IN), jnp.float32),
               pltpu.SemaphoreType.DMA,
               pltpu.SemaphoreType.DMA]
        ),
    )
    return f(xP, srcp, dstp, ewp, zrow)


def _edge_agg(x, edge_index, edge_features):
    i32 = jnp.int32
    src = edge_index[:, 0, :].astype(i32)
    dst = edge_index[:, 1, :].astype(i32)
    ew = edge_features[:, :, 0]
    padE = [(0, 0), (0, E_PAD - E)]
    sh3 = (T, E_PAD // 128, 128)
    srcp = jnp.pad(src, padE).reshape(sh3)
    dstp = jnp.pad(dst, padE).reshape(sh3)
    ewp = jnp.pad(ew, padE).reshape(sh3)
    xP = jnp.pad(jnp.transpose(x, (1, 0, 2)),
                 [(0, 0), (0, N_PAD - N), (0, 0)])
    zrow = jnp.zeros((_NROW, D_IN), jnp.float32)
    return _edge_call(xP, srcp, dstp, ewp, zrow)


@jax.jit
def kernel(x, edge_index, edge_features, params):
    agg = _edge_agg(x, edge_index, edge_features)
    dense_params = _prep_dense_params(params)
    pad = [(0, 0), (0, 0), (0, N_PAD - N)]
    xT = jnp.pad(jnp.transpose(x, (1, 2, 0)), pad)
    aggT = jnp.transpose(agg, (0, 1, 3, 2))
    outT, mwT = _dense_call(xT, aggT, dense_params)
    out = jnp.transpose(outT, (2, 0, 1))[:N]
    mw = mwT.T[:N]
    return (out, mw)
